# baseline (speedup 1.0000x reference)
import jax
import jax.numpy as jnp
from jax import lax
from jax.experimental import pallas as pl
from jax.experimental.pallas import tpu as pltpu

N_FACTORS = 64
BATCH = 16384
BLOCK = 1024
GRID = BATCH // BLOCK


def _body(uidx, iidx, ut_hbm, it_hbm, out_ref, stage, ubuf, ibuf, sems):
    b = pl.program_id(0)
    base = b * BLOCK

    def fire(r, _):
        pltpu.async_copy(ut_hbm.at[uidx[r]], ubuf.at[r], sems.at[0])
        pltpu.async_copy(it_hbm.at[iidx[r]], stage.at[base + r], sems.at[1])
        return 0

    lax.fori_loop(0, BLOCK, fire, 0, unroll=8)

    pltpu.make_async_copy(ut_hbm.at[pl.ds(0, BLOCK)], ubuf, sems.at[0]).wait()
    pltpu.make_async_copy(
        it_hbm.at[pl.ds(0, BLOCK)], stage.at[pl.ds(0, BLOCK)], sems.at[1]
    ).wait()

    cp = pltpu.make_async_copy(stage.at[pl.ds(base, BLOCK)], ibuf, sems.at[2])
    cp.start()
    cp.wait()

    out_ref[...] = jnp.sum(ubuf[...] * ibuf[...], axis=1)


@jax.jit
def _mf(users, items, user_table, item_table):
    f = pl.pallas_call(
        _body,
        grid=(GRID,),
        in_specs=[
            pl.BlockSpec((BLOCK,), lambda b: (b,), memory_space=pltpu.SMEM),
            pl.BlockSpec((BLOCK,), lambda b: (b,), memory_space=pltpu.SMEM),
            pl.BlockSpec(memory_space=pltpu.HBM),
            pl.BlockSpec(memory_space=pltpu.HBM),
        ],
        out_specs=[
            pl.BlockSpec((BLOCK,), lambda b: (b,)),
            pl.BlockSpec(memory_space=pltpu.HBM),
        ],
        out_shape=[
            jax.ShapeDtypeStruct((BATCH,), jnp.float32),
            jax.ShapeDtypeStruct((BATCH, N_FACTORS), jnp.float32),
        ],
        scratch_shapes=[
            pltpu.VMEM((BLOCK, N_FACTORS), jnp.float32),
            pltpu.VMEM((BLOCK, N_FACTORS), jnp.float32),
            pltpu.SemaphoreType.DMA((3,)),
        ],
    )
    out, _ = f(users, items, user_table, item_table)
    return out


def kernel(users, items, user_table, item_table):
    return _mf(users, items, user_table, item_table)
